# baseline (speedup 1.0000x reference)
import functools

import jax
import jax.numpy as jnp
from jax.experimental import pallas as pl
from jax.experimental.pallas import tpu as pltpu

_ROWS_PER_BLOCK = 256


def _tc_body(x_ref, w_ref, o_ref):
    x = x_ref[...].astype(jnp.int32)
    w = w_ref[...]
    oh = (x[:, :, None] == jax.lax.broadcasted_iota(jnp.int32, (1, 1, 8), 2))
    oh = oh.astype(jnp.float32)
    o_ref[...] = jax.lax.dot_general(
        oh, w, (((2,), (0,)), ((), ())),
        preferred_element_type=jnp.float32)


def kernel(X, W):
    n, m = X.shape
    f = W.shape[1]
    b = _ROWS_PER_BLOCK
    w8 = jnp.zeros((8, f), jnp.float32).at[:5].set(W)
    grid = (n // b,)
    out = pl.pallas_call(
        _tc_body,
        grid=grid,
        in_specs=[
            pl.BlockSpec((b, m), lambda i: (i, 0)),
            pl.BlockSpec((8, f), lambda i: (0, 0)),
        ],
        out_specs=pl.BlockSpec((b, m, f), lambda i: (i, 0, 0)),
        out_shape=jax.ShapeDtypeStruct((n, m, f), jnp.float32),
    )(X.astype(jnp.int32), w8)
    return out


# speedup vs baseline: 1.2481x; 1.2481x over previous
import jax
import jax.numpy as jnp
from jax.experimental import pallas as pl

_ROWS_PER_BLOCK = 256


def _tc_body(x_ref, wt_ref, o_ref):
    b = x_ref.shape[0]
    mf = o_ref.shape[1]
    x = x_ref[...].astype(jnp.int32)
    idx = jnp.arange(mf, dtype=jnp.int32) // 64
    ilo = jnp.broadcast_to(idx[None, :128 * 64], (b, 128 * 64))
    ihi = jnp.broadcast_to(idx[None, 128 * 64:] - 128, (b, mf - 128 * 64))
    xlo = jnp.take_along_axis(x[:, :128], ilo, axis=1)
    xhi = jnp.take_along_axis(x[:, 128:], ihi, axis=1)
    xr = jnp.concatenate([xlo, xhi], axis=1)
    wt = wt_ref[...]
    acc = jnp.broadcast_to(wt[4][None, :], (b, mf))
    acc = jnp.where(xr == 3, wt[3][None, :], acc)
    acc = jnp.where(xr == 2, wt[2][None, :], acc)
    acc = jnp.where(xr == 1, wt[1][None, :], acc)
    acc = jnp.where(xr == 0, wt[0][None, :], acc)
    o_ref[...] = acc


def kernel(X, W):
    n, m = X.shape
    f = W.shape[1]
    mf = m * f
    b = _ROWS_PER_BLOCK
    wt = jnp.tile(W, (1, m))
    out2d = pl.pallas_call(
        _tc_body,
        grid=(n // b,),
        in_specs=[
            pl.BlockSpec((b, m), lambda i: (i, 0)),
            pl.BlockSpec((5, mf), lambda i: (0, 0)),
        ],
        out_specs=pl.BlockSpec((b, mf), lambda i: (i, 0)),
        out_shape=jax.ShapeDtypeStruct((n, mf), jnp.float32),
    )(X.astype(jnp.int32), wt)
    return out2d.reshape(n, m, f)


# speedup vs baseline: 1.5920x; 1.2755x over previous
import jax
import jax.numpy as jnp
from jax.experimental import pallas as pl

_ROWS_PER_BLOCK = 256


def _tc_body(x_ref, wp_ref, o_ref):
    b = x_ref.shape[0]
    m = x_ref.shape[1]
    x = x_ref[...].astype(jnp.int32)
    wp = wp_ref[...]
    lane16 = jax.lax.broadcasted_iota(jnp.int32, (b, 16), 1)
    lane8 = lane16 & 7
    half = lane16 >> 3
    xlo = x[:, :128]
    xhi = x[:, 128:]
    for c in range(m // 2):
        if 2 * c + 1 < 128:
            idx = half + (2 * c)
            xc = jnp.take_along_axis(xlo, idx, axis=1)
        else:
            idx = half + (2 * c - 128)
            xc = jnp.take_along_axis(xhi, idx, axis=1)
        e2 = jnp.where(xc == lane8, 1.0, 0.0)
        outc = jax.lax.dot_general(
            e2, wp, (((1,), (0,)), ((), ())),
            preferred_element_type=jnp.float32)
        o_ref[:, 128 * c:128 * (c + 1)] = outc


def kernel(X, W):
    n, m = X.shape
    f = W.shape[1]
    mf = m * f
    b = _ROWS_PER_BLOCK
    wp = jnp.zeros((16, 2 * f), jnp.float32)
    wp = wp.at[:5, :f].set(W).at[8:13, f:].set(W)
    out2d = pl.pallas_call(
        _tc_body,
        grid=(n // b,),
        in_specs=[
            pl.BlockSpec((b, m), lambda i: (i, 0)),
            pl.BlockSpec((16, 2 * f), lambda i: (0, 0)),
        ],
        out_specs=pl.BlockSpec((b, mf), lambda i: (i, 0)),
        out_shape=jax.ShapeDtypeStruct((n, mf), jnp.float32),
    )(X.astype(jnp.int32), wp)
    return out2d.reshape(n, m, f)
